# baseline (speedup 1.0000x reference)
import jax
import jax.numpy as jnp
from jax import lax
from jax.experimental import pallas as pl
from jax.experimental.pallas import tpu as pltpu
from jax.experimental.pallas import tpu_sc as plsc

VOCAB = 1000000
DIM = 64
DIMP = 128
BATCH = 16384

NC = 2
NS = 16
L = 16
NW = NC * NS
B_PER_W = BATCH // NW
CH = 128
NCH = B_PER_W // CH
GPC = CH // L

_MESH = dict(core_axis_name="c", subcore_axis_name="s",
             num_cores=NC, num_subcores=NS)
_PARAMS = pltpu.CompilerParams(
    needs_layout_passes=False, use_tc_tiling_on_sc=True)


def _gather_body(idx_hbm, tbl, rows_hbm, iv, buf0, buf1, sems):
    wid = lax.axis_index("s") * NC + lax.axis_index("c")
    base = wid * B_PER_W
    pltpu.sync_copy(idx_hbm.at[pl.ds(base, B_PER_W)], iv)
    bufs = (buf0, buf1)

    def copy(ch, b):
        sl = iv.at[pl.ds(ch * CH, CH)]
        return pltpu.make_async_copy(tbl.at[sl], bufs[b], sems.at[b])

    copy(0, 0).start()
    copy(1, 1).start()
    for ch in range(NCH):
        b = ch % 2
        copy(ch, b).wait()
        pltpu.sync_copy(bufs[b], rows_hbm.at[pl.ds(base + ch * CH, CH)])
        if ch + 2 < NCH:
            copy(ch + 2, b).start()


def _dot_body(xidx_hbm, ctx_tbl, crows_hbm, out_hbm,
              xiv, out_v, part, cbuf0, cbuf1, xbuf0, xbuf1, sems):
    wid = lax.axis_index("s") * NC + lax.axis_index("c")
    base = wid * B_PER_W
    pltpu.sync_copy(xidx_hbm.at[pl.ds(base, B_PER_W)], xiv)

    cbufs = (cbuf0, cbuf1)
    xbufs = (xbuf0, xbuf1)

    def copies(ch, b):
        xsl = xiv.at[pl.ds(ch * CH, CH)]
        return (pltpu.make_async_copy(
                    crows_hbm.at[pl.ds(base + ch * CH, CH)],
                    cbufs[b], sems.at[b]),
                pltpu.make_async_copy(ctx_tbl.at[xsl], xbufs[b],
                                      sems.at[2 + b]))

    for b in range(2):
        for cp in copies(b, b):
            cp.start()

    col0 = lax.iota(jnp.int32, L) * L

    for ch in range(NCH):
        b = ch % 2
        for cp in copies(ch, b):
            cp.wait()

        def group(g, carry, b=b, ch=ch):
            for r in range(L):
                row = g * L + r
                p = (cbufs[b][row, pl.ds(0, L)] *
                     xbufs[b][row, pl.ds(0, L)])
                for j in range(1, DIM // L):
                    sl = pl.ds(j * L, L)
                    p = p + cbufs[b][row, sl] * xbufs[b][row, sl]
                part[pl.ds(r * L, L)] = p
            acc = plsc.load_gather(part, [col0])
            for c in range(1, L):
                acc = acc + plsc.load_gather(part, [col0 + c])
            out_v[pl.ds(ch * CH + g * L, L)] = acc
            return carry

        lax.fori_loop(0, GPC, group, 0)

        if ch + 2 < NCH:
            for cp in copies(ch + 2, b):
                cp.start()

    pltpu.sync_copy(out_v, out_hbm.at[pl.ds(base, B_PER_W)])


@jax.jit
def _scores(cidx, xidx, ctr_p, ctx_p):
    crows = pl.kernel(
        _gather_body,
        out_type=jax.ShapeDtypeStruct((BATCH, DIMP), jnp.float32),
        mesh=plsc.VectorSubcoreMesh(**_MESH),
        scratch_types=[
            pltpu.VMEM((B_PER_W,), jnp.int32),
            pltpu.VMEM((CH, DIMP), jnp.float32),
            pltpu.VMEM((CH, DIMP), jnp.float32),
            pltpu.SemaphoreType.DMA((2,)),
        ],
        compiler_params=_PARAMS,
    )(cidx, ctr_p)
    return pl.kernel(
        _dot_body,
        out_type=jax.ShapeDtypeStruct((BATCH,), jnp.float32),
        mesh=plsc.VectorSubcoreMesh(**_MESH),
        scratch_types=[
            pltpu.VMEM((B_PER_W,), jnp.int32),
            pltpu.VMEM((B_PER_W,), jnp.float32),
            pltpu.VMEM((L * L,), jnp.float32),
            pltpu.VMEM((CH, DIMP), jnp.float32),
            pltpu.VMEM((CH, DIMP), jnp.float32),
            pltpu.VMEM((CH, DIMP), jnp.float32),
            pltpu.VMEM((CH, DIMP), jnp.float32),
            pltpu.SemaphoreType.DMA((4,)),
        ],
        compiler_params=_PARAMS,
    )(xidx, ctx_p, crows)


def kernel(center_words, context_words, center_table, context_table):
    cidx = center_words.astype(jnp.int32)
    xidx = context_words.astype(jnp.int32)
    ctr_p = jnp.pad(center_table, ((0, 0), (0, DIMP - DIM)))
    ctx_p = jnp.pad(context_table, ((0, 0), (0, DIMP - DIM)))
    return _scores(cidx, xidx, ctr_p, ctx_p)
